# baseline (speedup 1.0000x reference)
import jax
import jax.numpy as jnp
from jax import lax
from jax.experimental import pallas as pl
from jax.experimental.pallas import tpu as pltpu
from jax.experimental.pallas import tpu_sc as plsc

N = 10000
E = 320000
B = 16
D = 128
O = 128

NC = 2
NS = 16
NW = NC * NS
EW = E // NW
K = 80
CHP = 128
GRP = 8
NG = CHP // GRP
EWP = CHP * K
PAD = EWP - EW
NP = 10112
RPS = NP // NS

_GATHER_DNUMS = lax.GatherDimensionNumbers(
    offset_dims=(), collapsed_slice_dims=(0,), start_index_map=(0,))


def _film_body(nf_ref, cond_ref, bid_ref, wc_ref, bc_ref, wl_ref, ew_ref,
               ep_ref, h_ref, w_ref):
    cond = cond_ref[...]
    wc = wc_ref[...]
    gb = lax.dot_general(cond, wc, (((1,), (1,)), ((), ())),
                         preferred_element_type=jnp.float32)
    gb = gb + jnp.concatenate([bc_ref[0:1, :], bc_ref[1:2, :]], axis=1)
    gamma = gb[:, :O] + 1.0
    beta = gb[:, O:]

    h = lax.dot_general(nf_ref[...], wl_ref[...], (((1,), (1,)), ((), ())),
                        preferred_element_type=jnp.float32)
    mu = jnp.mean(h, axis=1, keepdims=True)
    d = h - mu
    var = jnp.mean(d * d, axis=1, keepdims=True)
    hn = d * lax.rsqrt(var + 1e-5)

    bid = bid_ref[...]
    oh = (bid == lax.broadcasted_iota(jnp.int32, (1, B), 1))
    oh = oh.astype(jnp.float32)
    g = jnp.dot(oh, gamma, preferred_element_type=jnp.float32)
    b = jnp.dot(oh, beta, preferred_element_type=jnp.float32)
    h_ref[...] = jnp.maximum(hn * g + b, 0.0)

    w_ref[...] = ew_ref[...] * ep_ref[...]


_BN = 1000
_ER = 320
_BE = _ER // 10

_film_call = pl.pallas_call(
    _film_body,
    grid=(N // _BN,),
    in_specs=[
        pl.BlockSpec((_BN, D), lambda i: (i, 0)),
        pl.BlockSpec((B, D), lambda i: (0, 0)),
        pl.BlockSpec((_BN, 1), lambda i: (i, 0)),
        pl.BlockSpec((2 * O, D), lambda i: (0, 0)),
        pl.BlockSpec((2, O), lambda i: (0, 0)),
        pl.BlockSpec((O, D), lambda i: (0, 0)),
        pl.BlockSpec((_BE, 1000), lambda i: (i, 0)),
        pl.BlockSpec((_BE, 1000), lambda i: (i, 0)),
    ],
    out_specs=[
        pl.BlockSpec((_BN, O), lambda i: (i, 0)),
        pl.BlockSpec((_BE, 1000), lambda i: (i, 0)),
    ],
    out_shape=[
        jax.ShapeDtypeStruct((N, O), jnp.float32),
        jax.ShapeDtypeStruct((_ER, 1000), jnp.float32),
    ],
)


def _mp_body(h_hbm, jid_hbm, iid_hbm, w_hbm, out_hbm,
             jid_v, iid_v, w_v,
             rows0, rows1, stage0, stage1,
             sem_g0, sem_g1, sem_s0, sem_s1, sem_m, accum):
    c = lax.axis_index("c")
    s = lax.axis_index("s")

    zero16 = jnp.zeros((16,), jnp.float32)

    @pl.loop(0, K)
    def _zero_rows(r):
        for f in range(8):
            stage0[r, pl.ds(f * 16, 16)] = zero16

    base = s * RPS
    for kk in range(RPS // K):
        pltpu.sync_copy(stage0.at[pl.ds(0, K)],
                        accum.at[pl.ds(base + kk * K, K)])
    pltpu.sync_copy(stage0.at[pl.ds(0, RPS - (RPS // K) * K)],
                    accum.at[pl.ds(base + (RPS // K) * K,
                                   RPS - (RPS // K) * K)])
    plsc.subcore_barrier()

    def scale(src, dst, wrow):
        @pl.loop(0, K // 16)
        def _grp(g):
            w16 = wrow[pl.ds(g * 16, 16)]

            @pl.loop(0, 16)
            def _lane(le):
                splat = lax.gather(
                    w16, jnp.full((16, 1), le, jnp.int32), _GATHER_DNUMS,
                    (1,), mode=lax.GatherScatterMode.PROMISE_IN_BOUNDS)
                e = g * 16 + le
                for f in range(8):
                    dst[e, pl.ds(f * 16, 16)] = (
                        src[e, pl.ds(f * 16, 16)] * splat)

    def fire_meta(g):
        off = pl.multiple_of(g * GRP, 8)
        pltpu.async_copy(jid_hbm.at[c, s, pl.ds(off, GRP)],
                         jid_v.at[g % 2], sem_m)
        pltpu.async_copy(iid_hbm.at[c, s, pl.ds(off, GRP)],
                         iid_v.at[g % 3], sem_m)
        pltpu.async_copy(w_hbm.at[c, s, pl.ds(off, GRP)],
                         w_v.at[g % 2], sem_m)

    def wait_meta():
        pltpu.make_async_copy(jid_hbm.at[c, s, pl.ds(0, GRP)],
                              jid_v.at[0], sem_m).wait()
        pltpu.make_async_copy(iid_hbm.at[c, s, pl.ds(0, GRP)],
                              iid_v.at[0], sem_m).wait()
        pltpu.make_async_copy(w_hbm.at[c, s, pl.ds(0, GRP)],
                              w_v.at[0], sem_m).wait()

    def fire_g(dst, idx_row, sem):
        pltpu.async_copy(h_hbm.at[idx_row], dst, sem)

    def wait_g(dst, idx_row, sem):
        pltpu.make_async_copy(h_hbm.at[idx_row], dst, sem).wait()

    def fire_sc(src, idx_row, sem):
        pltpu.async_copy(src, accum.at[idx_row], sem, add=True)

    def wait_sc(src, idx_row, sem):
        pltpu.make_async_copy(src, accum.at[idx_row], sem).wait()

    fire_meta(0)
    fire_meta(1)
    wait_meta()
    fire_g(rows0, jid_v.at[0, 0], sem_g0)

    @pl.loop(0, NG)
    def _group(g):
        slot = g % 2
        islot = g % 3
        nslot = 1 - slot

        @pl.loop(0, GRP // 2)
        def _pair(t):
            k0 = 2 * t

            fire_g(rows1, jid_v.at[slot, k0 + 1], sem_g1)

            @pl.when(g + t > 0)
            def _w0():
                wait_sc(stage0, iid_v.at[islot, k0], sem_s0)

            wait_g(rows0, jid_v.at[slot, k0], sem_g0)
            scale(rows0, stage0, w_v.at[slot, k0])
            fire_sc(stage0, iid_v.at[islot, k0], sem_s0)

            @pl.when(t < GRP // 2 - 1)
            def _g_same():
                fire_g(rows0, jid_v.at[slot, k0 + 2], sem_g0)

            @pl.when((t == GRP // 2 - 1) & (g < NG - 1))
            def _g_next():
                wait_meta()
                fire_g(rows0, jid_v.at[nslot, 0], sem_g0)

            @pl.when(g + t > 0)
            def _w1():
                wait_sc(stage1, iid_v.at[islot, k0 + 1], sem_s1)

            wait_g(rows1, jid_v.at[slot, k0 + 1], sem_g1)
            scale(rows1, stage1, w_v.at[slot, k0 + 1])
            fire_sc(stage1, iid_v.at[islot, k0 + 1], sem_s1)

            @pl.when((t == GRP // 2 - 1) & (g < NG - 2))
            def _m_next():
                fire_meta(g + 2)

    wait_sc(stage0, iid_v.at[0, 0], sem_s0)
    wait_sc(stage1, iid_v.at[0, 0], sem_s1)

    plsc.subcore_barrier()

    pltpu.sync_copy(accum.at[pl.ds(base, RPS)],
                    out_hbm.at[c, pl.ds(base, RPS)])


_mp_call = pl.kernel(
    _mp_body,
    out_type=jax.ShapeDtypeStruct((NC, NP, O), jnp.float32),
    mesh=plsc.VectorSubcoreMesh(core_axis_name="c", subcore_axis_name="s"),
    scratch_types=[
        pltpu.VMEM((2, GRP, K), jnp.int32),
        pltpu.VMEM((3, GRP, K), jnp.int32),
        pltpu.VMEM((2, GRP, K), jnp.float32),
        pltpu.VMEM((K, O), jnp.float32),
        pltpu.VMEM((K, O), jnp.float32),
        pltpu.VMEM((K, O), jnp.float32),
        pltpu.VMEM((K, O), jnp.float32),
        pltpu.SemaphoreType.DMA,
        pltpu.SemaphoreType.DMA,
        pltpu.SemaphoreType.DMA,
        pltpu.SemaphoreType.DMA,
        pltpu.SemaphoreType.DMA,
        pltpu.VMEM_SHARED((NP, O), jnp.float32),
    ],
)


def _fin_body(p_ref, o_ref):
    p = p_ref[...]
    o_ref[...] = jnp.maximum(p[0, :N, :] + p[1, :N, :], 0.0)


_fin_call = pl.pallas_call(
    _fin_body,
    in_specs=[pl.BlockSpec((NC, NP, O), lambda: (0, 0, 0))],
    out_specs=pl.BlockSpec((N, O), lambda: (0, 0)),
    out_shape=jax.ShapeDtypeStruct((N, O), jnp.float32),
)


def kernel(node_feats, cond_feats, batch_ids, edge_weights, edge_params,
           node_j_ids, node_i_ids, W_cond, b_cond, W_lin):
    bid2 = batch_ids.reshape(N, 1)
    bc2 = b_cond.reshape(2, O)
    ew2 = edge_weights.reshape(_ER, 1000)
    ep2 = edge_params.reshape(_ER, 1000)

    h, w = _film_call(node_feats, cond_feats, bid2, W_cond, bc2, W_lin,
                      ew2, ep2)

    spread = (jnp.arange(NW * PAD, dtype=jnp.int32) % N).reshape(NW, PAD)
    wp = jnp.concatenate(
        [w.reshape(NW, EW), jnp.zeros((NW, PAD), jnp.float32)],
        axis=1).reshape(NC, NS, CHP, K)
    jp = jnp.concatenate([node_j_ids.reshape(NW, EW), spread],
                         axis=1).reshape(NC, NS, CHP, K)
    ip = jnp.concatenate([node_i_ids.reshape(NW, EW), spread],
                         axis=1).reshape(NC, NS, CHP, K)

    partials = _mp_call(h, jp, ip, wp)
    return _fin_call(partials)


# speedup vs baseline: 2.1271x; 2.1271x over previous
import jax
import jax.numpy as jnp
from jax import lax
from jax.experimental import pallas as pl
from jax.experimental.pallas import tpu as pltpu
from jax.experimental.pallas import tpu_sc as plsc

N = 10000
E = 320000
B = 16
D = 128
O = 128

NC = 2
NS = 16
NW = NC * NS
EW = E // NW
K = 128
CHP = 80
GRP = 8
NG = CHP // GRP
EWP = CHP * K
PAD = EWP - EW
NP = 10240
RPS = NP // NS

_GATHER_DNUMS = lax.GatherDimensionNumbers(
    offset_dims=(), collapsed_slice_dims=(0,), start_index_map=(0,))


def _film_body(nf_ref, cond_ref, bid_ref, wc_ref, bc_ref, wl_ref, ew_ref,
               ep_ref, h_ref, w_ref):
    cond = cond_ref[...]
    wc = wc_ref[...]
    gb = lax.dot_general(cond, wc, (((1,), (1,)), ((), ())),
                         preferred_element_type=jnp.float32)
    gb = gb + jnp.concatenate([bc_ref[0:1, :], bc_ref[1:2, :]], axis=1)
    gamma = gb[:, :O] + 1.0
    beta = gb[:, O:]

    h = lax.dot_general(nf_ref[...], wl_ref[...], (((1,), (1,)), ((), ())),
                        preferred_element_type=jnp.float32)
    mu = jnp.mean(h, axis=1, keepdims=True)
    d = h - mu
    var = jnp.mean(d * d, axis=1, keepdims=True)
    hn = d * lax.rsqrt(var + 1e-5)

    bid = bid_ref[...]
    oh = (bid == lax.broadcasted_iota(jnp.int32, (1, B), 1))
    oh = oh.astype(jnp.float32)
    g = jnp.dot(oh, gamma, preferred_element_type=jnp.float32)
    b = jnp.dot(oh, beta, preferred_element_type=jnp.float32)
    h_ref[...] = jnp.maximum(hn * g + b, 0.0)

    w_ref[...] = ew_ref[...] * ep_ref[...]


_BN = 1000
_ER = 320
_BE = _ER // 10

_film_call = pl.pallas_call(
    _film_body,
    grid=(N // _BN,),
    in_specs=[
        pl.BlockSpec((_BN, D), lambda i: (i, 0)),
        pl.BlockSpec((B, D), lambda i: (0, 0)),
        pl.BlockSpec((_BN, 1), lambda i: (i, 0)),
        pl.BlockSpec((2 * O, D), lambda i: (0, 0)),
        pl.BlockSpec((2, O), lambda i: (0, 0)),
        pl.BlockSpec((O, D), lambda i: (0, 0)),
        pl.BlockSpec((_BE, 1000), lambda i: (i, 0)),
        pl.BlockSpec((_BE, 1000), lambda i: (i, 0)),
    ],
    out_specs=[
        pl.BlockSpec((_BN, O), lambda i: (i, 0)),
        pl.BlockSpec((_BE, 1000), lambda i: (i, 0)),
    ],
    out_shape=[
        jax.ShapeDtypeStruct((N, O), jnp.float32),
        jax.ShapeDtypeStruct((_ER, 1000), jnp.float32),
    ],
)


def _mp_body(h_hbm, jid_hbm, iid_hbm, w_hbm, out_hbm,
             jid_v, iid_v, w_v, rows_a, rows_b, sem_a, sem_b, sem_m, accum):
    c = lax.axis_index("c")
    s = lax.axis_index("s")

    zero16 = jnp.zeros((16,), jnp.float32)

    @pl.loop(0, K)
    def _zero_rows(r):
        for f in range(8):
            rows_a[r, pl.ds(f * 16, 16)] = zero16

    base = s * RPS
    for kk in range(RPS // K):
        pltpu.sync_copy(rows_a.at[pl.ds(0, K)],
                        accum.at[pl.ds(base + kk * K, K)])
    plsc.subcore_barrier()

    def scale(rows_v, slot, kk):
        @pl.loop(0, K // 16)
        def _group(g):
            w16 = w_v[slot, kk, pl.ds(g * 16, 16)]
            for le in range(16):
                splat = lax.gather(
                    w16, jnp.full((16, 1), le, jnp.int32), _GATHER_DNUMS,
                    (1,), mode=lax.GatherScatterMode.PROMISE_IN_BOUNDS)
                e = g * 16 + le
                for f in range(8):
                    rows_v[e, pl.ds(f * 16, 16)] = (
                        rows_v[e, pl.ds(f * 16, 16)] * splat)

    def fire_meta(g, slot):
        pltpu.async_copy(jid_hbm.at[c, s, pl.ds(g * GRP, GRP)],
                         jid_v.at[slot], sem_m)
        pltpu.async_copy(iid_hbm.at[c, s, pl.ds(g * GRP, GRP)],
                         iid_v.at[slot], sem_m)
        pltpu.async_copy(w_hbm.at[c, s, pl.ds(g * GRP, GRP)],
                         w_v.at[slot], sem_m)

    def wait_meta(slot):
        pltpu.make_async_copy(jid_hbm.at[c, s, pl.ds(0, GRP)],
                              jid_v.at[slot], sem_m).wait()
        pltpu.make_async_copy(iid_hbm.at[c, s, pl.ds(0, GRP)],
                              iid_v.at[slot], sem_m).wait()
        pltpu.make_async_copy(w_hbm.at[c, s, pl.ds(0, GRP)],
                              w_v.at[slot], sem_m).wait()

    fire_meta(0, 0)
    for g in range(NG):
        slot = g % 2
        wait_meta(slot)
        if g + 1 < NG:
            fire_meta(g + 1, 1 - slot)

        pltpu.async_copy(h_hbm.at[jid_v.at[slot, 0]], rows_a, sem_a)

        @pl.loop(0, GRP // 2)
        def _pair(t):
            kk = t * 2
            pltpu.make_async_copy(h_hbm.at[jid_v.at[slot, kk]], rows_a,
                                  sem_a).wait()
            pltpu.async_copy(h_hbm.at[jid_v.at[slot, kk + 1]], rows_b, sem_b)
            scale(rows_a, slot, kk)
            pltpu.sync_copy(rows_a, accum.at[iid_v.at[slot, kk]], add=True)

            pltpu.make_async_copy(h_hbm.at[jid_v.at[slot, kk + 1]], rows_b,
                                  sem_b).wait()

            @pl.when(kk + 2 < GRP)
            def _prefetch():
                pltpu.async_copy(h_hbm.at[jid_v.at[slot, kk + 2]], rows_a,
                                 sem_a)

            scale(rows_b, slot, kk + 1)
            pltpu.sync_copy(rows_b, accum.at[iid_v.at[slot, kk + 1]],
                            add=True)

    plsc.subcore_barrier()

    pltpu.sync_copy(accum.at[pl.ds(base, RPS)],
                    out_hbm.at[c, pl.ds(base, RPS)])


_mp_call = pl.kernel(
    _mp_body,
    out_type=jax.ShapeDtypeStruct((NC, NP, O), jnp.float32),
    mesh=plsc.VectorSubcoreMesh(core_axis_name="c", subcore_axis_name="s"),
    scratch_types=[
        pltpu.VMEM((2, GRP, K), jnp.int32),
        pltpu.VMEM((2, GRP, K), jnp.int32),
        pltpu.VMEM((2, GRP, K), jnp.float32),
        pltpu.VMEM((K, O), jnp.float32),
        pltpu.VMEM((K, O), jnp.float32),
        pltpu.SemaphoreType.DMA,
        pltpu.SemaphoreType.DMA,
        pltpu.SemaphoreType.DMA,
        pltpu.VMEM_SHARED((NP, O), jnp.float32),
    ],
)


def _fin_body(p_ref, o_ref):
    p = p_ref[...]
    o_ref[...] = jnp.maximum(p[0, :N, :] + p[1, :N, :], 0.0)


_fin_call = pl.pallas_call(
    _fin_body,
    in_specs=[pl.BlockSpec((NC, NP, O), lambda: (0, 0, 0))],
    out_specs=pl.BlockSpec((N, O), lambda: (0, 0)),
    out_shape=jax.ShapeDtypeStruct((N, O), jnp.float32),
)


def kernel(node_feats, cond_feats, batch_ids, edge_weights, edge_params,
           node_j_ids, node_i_ids, W_cond, b_cond, W_lin):
    bid2 = batch_ids.reshape(N, 1)
    bc2 = b_cond.reshape(2, O)
    ew2 = edge_weights.reshape(_ER, 1000)
    ep2 = edge_params.reshape(_ER, 1000)

    h, w = _film_call(node_feats, cond_feats, bid2, W_cond, bc2, W_lin,
                      ew2, ep2)

    spread = (jnp.arange(NW * PAD, dtype=jnp.int32) % N).reshape(NW, PAD)
    wp = jnp.concatenate(
        [w.reshape(NW, EW), jnp.zeros((NW, PAD), jnp.float32)],
        axis=1).reshape(NC, NS, CHP, K)
    jp = jnp.concatenate([node_j_ids.reshape(NW, EW), spread],
                         axis=1).reshape(NC, NS, CHP, K)
    ip = jnp.concatenate([node_i_ids.reshape(NW, EW), spread],
                         axis=1).reshape(NC, NS, CHP, K)

    partials = _mp_call(h, jp, ip, wp)
    return _fin_call(partials)


# speedup vs baseline: 2.1281x; 1.0005x over previous
import jax
import jax.numpy as jnp
from jax import lax
from jax.experimental import pallas as pl
from jax.experimental.pallas import tpu as pltpu
from jax.experimental.pallas import tpu_sc as plsc

N = 10000
E = 320000
B = 16
D = 128
O = 128

NC = 2
NS = 16
NW = NC * NS
EW = E // NW
K = 128
CHP = 80
GRP = 8
NG = CHP // GRP
EWP = CHP * K
PAD = EWP - EW
NP = 10240
RPS = NP // NS

_GATHER_DNUMS = lax.GatherDimensionNumbers(
    offset_dims=(), collapsed_slice_dims=(0,), start_index_map=(0,))


def _film_body(nf_ref, cond_ref, bid_ref, wc_ref, bc_ref, wl_ref, ew_ref,
               ep_ref, h_ref, w_ref):
    cond = cond_ref[...]
    wc = wc_ref[...]
    gb = lax.dot_general(cond, wc, (((1,), (1,)), ((), ())),
                         preferred_element_type=jnp.float32)
    gb = gb + jnp.concatenate([bc_ref[0:1, :], bc_ref[1:2, :]], axis=1)
    gamma = gb[:, :O] + 1.0
    beta = gb[:, O:]

    h = lax.dot_general(nf_ref[...], wl_ref[...], (((1,), (1,)), ((), ())),
                        preferred_element_type=jnp.float32)
    mu = jnp.mean(h, axis=1, keepdims=True)
    d = h - mu
    var = jnp.mean(d * d, axis=1, keepdims=True)
    hn = d * lax.rsqrt(var + 1e-5)

    bid = bid_ref[...]
    oh = (bid == lax.broadcasted_iota(jnp.int32, (1, B), 1))
    oh = oh.astype(jnp.float32)
    g = jnp.dot(oh, gamma, preferred_element_type=jnp.float32)
    b = jnp.dot(oh, beta, preferred_element_type=jnp.float32)
    h_ref[...] = jnp.maximum(hn * g + b, 0.0)

    w_ref[...] = ew_ref[...] * ep_ref[...]


_BN = 1000
_ER = 320
_BE = _ER // 10

_film_call = pl.pallas_call(
    _film_body,
    grid=(N // _BN,),
    in_specs=[
        pl.BlockSpec((_BN, D), lambda i: (i, 0)),
        pl.BlockSpec((B, D), lambda i: (0, 0)),
        pl.BlockSpec((_BN, 1), lambda i: (i, 0)),
        pl.BlockSpec((2 * O, D), lambda i: (0, 0)),
        pl.BlockSpec((2, O), lambda i: (0, 0)),
        pl.BlockSpec((O, D), lambda i: (0, 0)),
        pl.BlockSpec((_BE, 1000), lambda i: (i, 0)),
        pl.BlockSpec((_BE, 1000), lambda i: (i, 0)),
    ],
    out_specs=[
        pl.BlockSpec((_BN, O), lambda i: (i, 0)),
        pl.BlockSpec((_BE, 1000), lambda i: (i, 0)),
    ],
    out_shape=[
        jax.ShapeDtypeStruct((N, O), jnp.float32),
        jax.ShapeDtypeStruct((_ER, 1000), jnp.float32),
    ],
)


def _mp_body(h_hbm, jid_hbm, iid_hbm, w_hbm, out_hbm,
             jid_v, iid_v, w_v, rows_a, rows_b, sem_a, sem_b, sem_m,
             sem_sa, sem_sb, accum):
    c = lax.axis_index("c")
    s = lax.axis_index("s")

    zero16 = jnp.zeros((16,), jnp.float32)

    @pl.loop(0, K)
    def _zero_rows(r):
        for f in range(8):
            rows_a[r, pl.ds(f * 16, 16)] = zero16

    base = s * RPS
    for kk in range(RPS // K):
        pltpu.sync_copy(rows_a.at[pl.ds(0, K)],
                        accum.at[pl.ds(base + kk * K, K)])
    plsc.subcore_barrier()

    def scale(rows_v, slot, kk):
        @pl.loop(0, K // 16)
        def _group(g):
            w16 = w_v[slot, kk, pl.ds(g * 16, 16)]
            for le in range(16):
                splat = lax.gather(
                    w16, jnp.full((16, 1), le, jnp.int32), _GATHER_DNUMS,
                    (1,), mode=lax.GatherScatterMode.PROMISE_IN_BOUNDS)
                e = g * 16 + le
                for f in range(8):
                    rows_v[e, pl.ds(f * 16, 16)] = (
                        rows_v[e, pl.ds(f * 16, 16)] * splat)

    def fire_meta(g, slot):
        pltpu.async_copy(jid_hbm.at[c, s, pl.ds(g * GRP, GRP)],
                         jid_v.at[slot], sem_m)
        pltpu.async_copy(iid_hbm.at[c, s, pl.ds(g * GRP, GRP)],
                         iid_v.at[slot], sem_m)
        pltpu.async_copy(w_hbm.at[c, s, pl.ds(g * GRP, GRP)],
                         w_v.at[slot], sem_m)

    def wait_meta(slot):
        pltpu.make_async_copy(jid_hbm.at[c, s, pl.ds(0, GRP)],
                              jid_v.at[slot], sem_m).wait()
        pltpu.make_async_copy(iid_hbm.at[c, s, pl.ds(0, GRP)],
                              iid_v.at[slot], sem_m).wait()
        pltpu.make_async_copy(w_hbm.at[c, s, pl.ds(0, GRP)],
                              w_v.at[slot], sem_m).wait()

    def wait_sc(rows_v, sem):
        pltpu.make_async_copy(rows_v, accum.at[iid_v.at[0, 0]], sem).wait()

    fire_meta(0, 0)
    for g in range(NG):
        slot = g % 2
        wait_meta(slot)
        if g > 0:
            wait_sc(rows_a, sem_sa)
            wait_sc(rows_b, sem_sb)
        if g + 1 < NG:
            fire_meta(g + 1, 1 - slot)

        pltpu.async_copy(h_hbm.at[jid_v.at[slot, 0]], rows_a, sem_a)

        @pl.loop(0, GRP // 2)
        def _pair(t):
            kk = t * 2
            pltpu.make_async_copy(h_hbm.at[jid_v.at[slot, kk]], rows_a,
                                  sem_a).wait()

            @pl.when(t > 0)
            def _wb():
                wait_sc(rows_b, sem_sb)

            pltpu.async_copy(h_hbm.at[jid_v.at[slot, kk + 1]], rows_b, sem_b)
            scale(rows_a, slot, kk)
            pltpu.async_copy(rows_a, accum.at[iid_v.at[slot, kk]], sem_sa,
                             add=True)

            pltpu.make_async_copy(h_hbm.at[jid_v.at[slot, kk + 1]], rows_b,
                                  sem_b).wait()

            @pl.when(kk + 2 < GRP)
            def _prefetch():
                wait_sc(rows_a, sem_sa)
                pltpu.async_copy(h_hbm.at[jid_v.at[slot, kk + 2]], rows_a,
                                 sem_a)

            scale(rows_b, slot, kk + 1)
            pltpu.async_copy(rows_b, accum.at[iid_v.at[slot, kk + 1]],
                             sem_sb, add=True)

    wait_sc(rows_a, sem_sa)
    wait_sc(rows_b, sem_sb)
    plsc.subcore_barrier()

    pltpu.sync_copy(accum.at[pl.ds(base, RPS)],
                    out_hbm.at[c, pl.ds(base, RPS)])


_mp_call = pl.kernel(
    _mp_body,
    out_type=jax.ShapeDtypeStruct((NC, NP, O), jnp.float32),
    mesh=plsc.VectorSubcoreMesh(core_axis_name="c", subcore_axis_name="s"),
    scratch_types=[
        pltpu.VMEM((2, GRP, K), jnp.int32),
        pltpu.VMEM((2, GRP, K), jnp.int32),
        pltpu.VMEM((2, GRP, K), jnp.float32),
        pltpu.VMEM((K, O), jnp.float32),
        pltpu.VMEM((K, O), jnp.float32),
        pltpu.SemaphoreType.DMA,
        pltpu.SemaphoreType.DMA,
        pltpu.SemaphoreType.DMA,
        pltpu.SemaphoreType.DMA,
        pltpu.SemaphoreType.DMA,
        pltpu.VMEM_SHARED((NP, O), jnp.float32),
    ],
)


def _fin_body(p_ref, o_ref):
    p = p_ref[...]
    o_ref[...] = jnp.maximum(p[0, :N, :] + p[1, :N, :], 0.0)


_fin_call = pl.pallas_call(
    _fin_body,
    in_specs=[pl.BlockSpec((NC, NP, O), lambda: (0, 0, 0))],
    out_specs=pl.BlockSpec((N, O), lambda: (0, 0)),
    out_shape=jax.ShapeDtypeStruct((N, O), jnp.float32),
)


def kernel(node_feats, cond_feats, batch_ids, edge_weights, edge_params,
           node_j_ids, node_i_ids, W_cond, b_cond, W_lin):
    bid2 = batch_ids.reshape(N, 1)
    bc2 = b_cond.reshape(2, O)
    ew2 = edge_weights.reshape(_ER, 1000)
    ep2 = edge_params.reshape(_ER, 1000)

    h, w = _film_call(node_feats, cond_feats, bid2, W_cond, bc2, W_lin,
                      ew2, ep2)

    spread = (jnp.arange(NW * PAD, dtype=jnp.int32) % N).reshape(NW, PAD)
    wp = jnp.concatenate(
        [w.reshape(NW, EW), jnp.zeros((NW, PAD), jnp.float32)],
        axis=1).reshape(NC, NS, CHP, K)
    jp = jnp.concatenate([node_j_ids.reshape(NW, EW), spread],
                         axis=1).reshape(NC, NS, CHP, K)
    ip = jnp.concatenate([node_i_ids.reshape(NW, EW), spread],
                         axis=1).reshape(NC, NS, CHP, K)

    partials = _mp_call(h, jp, ip, wp)
    return _fin_call(partials)
